# baseline (speedup 1.0000x reference)
import dataclasses
import functools

import jax
import jax.numpy as jnp
from jax import lax
from jax.experimental import pallas as pl
from jax.experimental.pallas import tpu as pltpu
from jax.experimental.pallas import tpu_sc as plsc

_NC = 2
_NS = 16
_NW = _NC * _NS
_W = 160
_K = 32
_NR = _W // _K
_VTXT = 100000


def _gather_sc(base_w, new_w, idx, n, d):
    b_per_w = n // _NW
    n_chunks = b_per_w // _W
    assert n_chunks % 2 == 0 and n_chunks >= 4
    mesh = plsc.VectorSubcoreMesh(core_axis_name="c", subcore_axis_name="s")

    cparams = pltpu.CompilerParams()
    if "needs_layout_passes" in pltpu.CompilerParams.__dataclass_fields__:
        cparams = dataclasses.replace(cparams, needs_layout_passes=False)

    @functools.partial(
        pl.kernel,
        out_type=jax.ShapeDtypeStruct((n, d), jnp.float32),
        mesh=mesh,
        compiler_params=cparams,
        scratch_types=[
            pltpu.VMEM((b_per_w,), jnp.int32),
            pltpu.VMEM((_W,), jnp.int32),
            pltpu.VMEM((_W,), jnp.int32),
            pltpu.VMEM((_W, d), jnp.float32),
            pltpu.VMEM((_W, d), jnp.float32),
            pltpu.VMEM((_W, d), jnp.float32),
            pltpu.VMEM((_W, d), jnp.float32),
            pltpu.VMEM((_W,), jnp.int32),
            pltpu.VMEM((_W,), jnp.int32),
            pltpu.VMEM((_W,), jnp.int32),
            pltpu.VMEM((_W,), jnp.int32),
            pltpu.SMEM((2,), jnp.int32),
            pltpu.SemaphoreType.DMA,
            pltpu.SemaphoreType.DMA,
            pltpu.SemaphoreType.DMA,
            pltpu.SemaphoreType.DMA,
            pltpu.SemaphoreType.DMA,
            pltpu.SemaphoreType.DMA,
        ],
    )
    def gather_kernel(base_hbm, new_hbm, idx_hbm, out_hbm, idx_v,
                      midx0, midx1, rows0, rows1, stage0, stage1,
                      fixpos0, fixpos1, fixid0, fixid1, nov,
                      gsem0, gsem1, fsem0, fsem1, osem0, osem1):
        wid = lax.axis_index("s") * _NC + lax.axis_index("c")
        wbase = wid * b_per_w
        pltpu.sync_copy(idx_hbm.at[pl.ds(wbase, b_per_w)], idx_v)

        midx = (midx0, midx1)
        rows = (rows0, rows1)
        stage = (stage0, stage1)
        fixpos = (fixpos0, fixpos1)
        fixid = (fixid0, fixid1)
        gsem = (gsem0, gsem1)
        fsem = (fsem0, fsem1)
        osem = (osem0, osem1)

        iota = lax.iota(jnp.int32, 16)

        for v in range(_W // 16):
            fixid0[pl.ds(v * 16, 16)] = iota + v * 16
            fixid1[pl.ds(v * 16, 16)] = iota + v * 16

        def compute(c, p):
            cnt = jnp.int32(0)
            for v in range(_W // 16):
                ids = idx_v[pl.ds(c * _W + v * 16, 16)]
                m = ids >= _VTXT
                m32 = m.astype(jnp.int32)
                midx[p][pl.ds(v * 16, 16)] = ids - m32 * _VTXT
                pos = plsc.cumsum(m32) - 1 + cnt
                plsc.store_scatter(fixpos[p], [pos], iota + v * 16, mask=m)
                plsc.store_scatter(fixid[p], [pos], ids - _VTXT, mask=m)
                cnt = cnt + jnp.sum(m32)
            return cnt

        def gstart(p):
            pltpu.async_copy(base_hbm.at[midx[p]], rows[p], gsem[p])

        def gwait(p):
            pltpu.make_async_copy(base_hbm.at[midx[p]], rows[p],
                                  gsem[p]).wait()

        def fstart(p, cnt):
            for r in range(_NR):
                @pl.when(cnt > r * _K)
                def _():
                    pltpu.async_copy(
                        new_hbm.at[fixid[p].at[pl.ds(r * _K, _K)]],
                        stage[p].at[pl.ds(r * _K, _K)], fsem[p])

        def fwait(p, cnt):
            for r in range(_NR):
                @pl.when(cnt > r * _K)
                def _():
                    pltpu.make_async_copy(
                        new_hbm.at[fixid[p].at[pl.ds(r * _K, _K)]],
                        stage[p].at[pl.ds(r * _K, _K)], fsem[p]).wait()

        def place(p, cnt):
            def body(j, carry):
                jb = jnp.full((16,), j, jnp.int32)
                posv = plsc.load_gather(fixpos[p], [jb])
                for k in range(d // 16):
                    col = iota + k * 16
                    val = plsc.load_gather(stage[p], [jb, col])
                    plsc.store_scatter(rows[p], [posv, col], val)
                return carry
            lax.fori_loop(0, cnt, body, jnp.int32(0))

        def ostart(c, p):
            pltpu.async_copy(rows[p], out_hbm.at[pl.ds(wbase + c * _W, _W)],
                             osem[p])

        def owait(p):
            pltpu.make_async_copy(rows[p], out_hbm.at[pl.ds(wbase, _W)],
                                  osem[p]).wait()

        n0 = compute(0, 0)
        nov[0] = n0
        gstart(0)
        fstart(0, n0)
        n1 = compute(1, 1)
        nov[1] = n1
        gstart(1)
        fstart(1, n1)

        @pl.loop(0, n_chunks - 2, step=2)
        def _(c):
            for p in range(2):
                ch = c + p
                cnt = nov[p]
                gwait(p)
                fwait(p, cnt)
                place(p, cnt)
                ostart(ch, p)
                n2 = compute(ch + 2, p)
                nov[p] = n2
                owait(p)
                gstart(p)
                fstart(p, n2)

        for p in range(2):
            ch = n_chunks - 2 + p
            cnt = nov[p]
            gwait(p)
            fwait(p, cnt)
            place(p, cnt)
            ostart(ch, p)
            owait(p)

    return gather_kernel(base_w, new_w, idx)


def kernel(input_ids, base_weight, new_weight):
    b, h = input_ids.shape
    d = base_weight.shape[1]
    idx = input_ids.reshape(-1).astype(jnp.int32)
    out = _gather_sc(base_weight, new_weight, idx, idx.shape[0], d)
    return out.reshape(b, h, d)


# speedup vs baseline: 2.0947x; 2.0947x over previous
import dataclasses
import functools

import jax
import jax.numpy as jnp
from jax import lax
from jax.experimental import pallas as pl
from jax.experimental.pallas import tpu as pltpu
from jax.experimental.pallas import tpu_sc as plsc

_NC = 2
_NS = 16
_NW = _NC * _NS
_W = 128
_K = 128
_VTXT = 100000


def _gather_sc(base_w, new_w, idx, n, d):
    b_per_w = n // _NW
    n_chunks = b_per_w // _W
    n_rounds = (b_per_w + _K - 1) // _K + 1
    assert n_chunks % 2 == 0 and n_chunks >= 4
    mesh = plsc.VectorSubcoreMesh(core_axis_name="c", subcore_axis_name="s")

    cparams = pltpu.CompilerParams()
    if "needs_layout_passes" in pltpu.CompilerParams.__dataclass_fields__:
        cparams = dataclasses.replace(cparams, needs_layout_passes=False)

    @functools.partial(
        pl.kernel,
        out_type=jax.ShapeDtypeStruct((n, d), jnp.float32),
        mesh=mesh,
        compiler_params=cparams,
        scratch_types=[
            pltpu.VMEM((b_per_w,), jnp.int32),
            pltpu.VMEM((_W,), jnp.int32),
            pltpu.VMEM((_W,), jnp.int32),
            pltpu.VMEM((_W, d), jnp.float32),
            pltpu.VMEM((_W, d), jnp.float32),
            pltpu.VMEM((_K, d), jnp.float32),
            pltpu.VMEM((n_rounds, _K), jnp.int32),
            pltpu.VMEM((n_rounds, _K), jnp.int32),
            pltpu.SMEM((2,), jnp.int32),
            pltpu.SemaphoreType.DMA,
            pltpu.SemaphoreType.DMA,
            pltpu.SemaphoreType.DMA,
            pltpu.SemaphoreType.DMA,
            pltpu.SemaphoreType.DMA,
        ],
    )
    def gather_kernel(base_hbm, new_hbm, idx_hbm, out_hbm, idx_v,
                      midx0, midx1, rows0, rows1, stage, fpos, fid, tot,
                      gsem0, gsem1, osem0, osem1, fsem):
        wid = lax.axis_index("s") * _NC + lax.axis_index("c")
        wbase = wid * b_per_w
        pltpu.sync_copy(idx_hbm.at[pl.ds(wbase, b_per_w)], idx_v)

        midx = (midx0, midx1)
        rows = (rows0, rows1)
        gsem = (gsem0, gsem1)
        osem = (osem0, osem1)

        iota = lax.iota(jnp.int32, 16)
        tot[0] = jnp.int32(0)

        def compute(c, p):
            cnt = tot[0]
            for v in range(_W // 16):
                ids = idx_v[pl.ds(c * _W + v * 16, 16)]
                m = ids >= _VTXT
                m32 = m.astype(jnp.int32)
                midx[p][pl.ds(v * 16, 16)] = ids - m32 * _VTXT
                pos = plsc.cumsum(m32) - 1 + cnt
                row = lax.shift_right_logical(pos, 7)
                col = lax.bitwise_and(pos, 127)
                gp = wbase + c * _W + v * 16 + iota
                plsc.store_scatter(fpos, [row, col], gp, mask=m)
                plsc.store_scatter(fid, [row, col], ids - _VTXT, mask=m)
                cnt = cnt + jnp.sum(m32)
            tot[0] = cnt

        def gstart(p):
            pltpu.async_copy(base_hbm.at[midx[p]], rows[p], gsem[p])

        def gwait(p):
            pltpu.make_async_copy(base_hbm.at[midx[p]], rows[p],
                                  gsem[p]).wait()

        def ostart(c, p):
            pltpu.async_copy(rows[p], out_hbm.at[pl.ds(wbase + c * _W, _W)],
                             osem[p])

        def owait(p):
            pltpu.make_async_copy(rows[p], out_hbm.at[pl.ds(wbase, _W)],
                                  osem[p]).wait()

        compute(0, 0)
        gstart(0)
        compute(1, 1)
        gstart(1)
        gwait(0)
        ostart(0, 0)

        @pl.loop(1, n_chunks - 1, step=2)
        def _(c):
            owait(0)
            compute(c + 1, 0)
            gstart(0)
            gwait(1)
            ostart(c, 1)
            owait(1)
            compute(c + 2, 1)
            gstart(1)
            gwait(0)
            ostart(c + 1, 0)

        gwait(1)
        ostart(n_chunks - 1, 1)
        owait(0)
        owait(1)

        cnt = tot[0]

        @pl.when(cnt > 0)
        def _():
            zeros = jnp.zeros((16,), jnp.int32)
            e0p = plsc.load_gather(fpos, [zeros, zeros])
            e0i = plsc.load_gather(fid, [zeros, zeros])
            for v in range(_K // 16):
                flat = cnt + v * 16 + iota
                row = lax.shift_right_logical(flat, 7)
                col = lax.bitwise_and(flat, 127)
                plsc.store_scatter(fpos, [row, col], e0p)
                plsc.store_scatter(fid, [row, col], e0i)

            nr = lax.shift_right_logical(cnt + (_K - 1), 7)

            def body(r, carry):
                pltpu.async_copy(new_hbm.at[fid.at[r]], stage, fsem).wait()
                pltpu.async_copy(stage, out_hbm.at[fpos.at[r]], fsem).wait()
                return carry

            lax.fori_loop(0, nr, body, jnp.int32(0))

    return gather_kernel(base_w, new_w, idx)


def kernel(input_ids, base_weight, new_weight):
    b, h = input_ids.shape
    d = base_weight.shape[1]
    idx = input_ids.reshape(-1).astype(jnp.int32)
    out = _gather_sc(base_weight, new_weight, idx, idx.shape[0], d)
    return out.reshape(b, h, d)


# speedup vs baseline: 2.1128x; 1.0086x over previous
import dataclasses
import functools

import jax
import jax.numpy as jnp
from jax import lax
from jax.experimental import pallas as pl
from jax.experimental.pallas import tpu as pltpu
from jax.experimental.pallas import tpu_sc as plsc

_NC = 2
_NS = 16
_NW = _NC * _NS
_W = 128
_K = 128
_VTXT = 100000


def _gather_sc(base_w, new_w, idx, n, d):
    b_per_w = n // _NW
    n_chunks = b_per_w // _W
    n_rounds = (b_per_w + _K - 1) // _K + 1
    assert n_chunks % 2 == 0 and n_chunks >= 4
    mesh = plsc.VectorSubcoreMesh(core_axis_name="c", subcore_axis_name="s")

    cparams = pltpu.CompilerParams()
    if "needs_layout_passes" in pltpu.CompilerParams.__dataclass_fields__:
        cparams = dataclasses.replace(cparams, needs_layout_passes=False)

    @functools.partial(
        pl.kernel,
        out_type=jax.ShapeDtypeStruct((n, d), jnp.float32),
        mesh=mesh,
        compiler_params=cparams,
        scratch_types=[
            pltpu.VMEM((b_per_w,), jnp.int32),
            pltpu.VMEM((_W,), jnp.int32),
            pltpu.VMEM((_W,), jnp.int32),
            pltpu.VMEM((_W, d), jnp.float32),
            pltpu.VMEM((_W, d), jnp.float32),
            pltpu.VMEM((n_rounds, _K), jnp.int32),
            pltpu.VMEM((n_rounds, _K), jnp.int32),
            pltpu.SMEM((2,), jnp.int32),
            pltpu.SemaphoreType.DMA,
            pltpu.SemaphoreType.DMA,
            pltpu.SemaphoreType.DMA,
            pltpu.SemaphoreType.DMA,
        ],
    )
    def gather_kernel(base_hbm, new_hbm, idx_hbm, out_hbm, idx_v,
                      midx0, midx1, rows0, rows1, fpos, fid, tot,
                      gsem0, gsem1, osem0, osem1):
        wid = lax.axis_index("s") * _NC + lax.axis_index("c")
        wbase = wid * b_per_w
        pltpu.sync_copy(idx_hbm.at[pl.ds(wbase, b_per_w)], idx_v)

        midx = (midx0, midx1)
        rows = (rows0, rows1)
        gsem = (gsem0, gsem1)
        osem = (osem0, osem1)

        iota = lax.iota(jnp.int32, 16)
        tot[0] = jnp.int32(0)

        def compute(c, p):
            cnt = tot[0]
            for v in range(_W // 16):
                ids = idx_v[pl.ds(c * _W + v * 16, 16)]
                m = ids >= _VTXT
                m32 = m.astype(jnp.int32)
                midx[p][pl.ds(v * 16, 16)] = ids - m32 * _VTXT
                pos = plsc.cumsum(m32) - 1 + cnt
                row = lax.shift_right_logical(pos, 7)
                col = lax.bitwise_and(pos, 127)
                gp = wbase + c * _W + v * 16 + iota
                plsc.store_scatter(fpos, [row, col], gp, mask=m)
                plsc.store_scatter(fid, [row, col], ids - _VTXT, mask=m)
                cnt = cnt + jnp.sum(m32)
            tot[0] = cnt

        def gstart(p):
            pltpu.async_copy(base_hbm.at[midx[p]], rows[p], gsem[p])

        def gwait(p):
            pltpu.make_async_copy(base_hbm.at[midx[p]], rows[p],
                                  gsem[p]).wait()

        def ostart(c, p):
            pltpu.async_copy(rows[p], out_hbm.at[pl.ds(wbase + c * _W, _W)],
                             osem[p])

        def owait(p):
            pltpu.make_async_copy(rows[p], out_hbm.at[pl.ds(wbase, _W)],
                                  osem[p]).wait()

        compute(0, 0)
        gstart(0)
        compute(1, 1)
        gstart(1)
        gwait(0)
        ostart(0, 0)

        @pl.loop(1, n_chunks - 1, step=2)
        def _(c):
            owait(0)
            compute(c + 1, 0)
            gstart(0)
            gwait(1)
            ostart(c, 1)
            owait(1)
            compute(c + 2, 1)
            gstart(1)
            gwait(0)
            ostart(c + 1, 0)

        gwait(1)
        ostart(n_chunks - 1, 1)
        owait(0)
        owait(1)

        cnt = tot[0]

        @pl.when(cnt > 0)
        def _():
            zeros = jnp.zeros((16,), jnp.int32)
            e0p = plsc.load_gather(fpos, [zeros, zeros])
            e0i = plsc.load_gather(fid, [zeros, zeros])
            for v in range(_K // 16):
                flat = cnt + v * 16 + iota
                row = lax.shift_right_logical(flat, 7)
                col = lax.bitwise_and(flat, 127)
                plsc.store_scatter(fpos, [row, col], e0p)
                plsc.store_scatter(fid, [row, col], e0i)

            nr = lax.shift_right_logical(cnt + (_K - 1), 7)

            def g2start(r, p):
                pltpu.async_copy(new_hbm.at[fid.at[r]], rows[p], gsem[p])

            def g2wait(p):
                pltpu.make_async_copy(new_hbm.at[pl.ds(0, _K)], rows[p],
                                      gsem[p]).wait()

            def s2start(r, p):
                pltpu.async_copy(rows[p], out_hbm.at[fpos.at[r]], osem[p])

            def s2wait(p):
                pltpu.make_async_copy(rows[p], out_hbm.at[pl.ds(wbase, _K)],
                                      osem[p]).wait()

            g2start(0, 0)

            def pair(i, carry):
                r0 = 2 * i
                r1 = r0 + 1
                g2wait(0)
                s2start(r0, 0)

                @pl.when(i > 0)
                def _():
                    s2wait(1)

                @pl.when(r1 < nr)
                def _():
                    g2start(r1, 1)
                    g2wait(1)
                    s2wait(0)

                    @pl.when(r1 + 1 < nr)
                    def _():
                        g2start(r1 + 1, 0)

                    s2start(r1, 1)

                return carry

            lax.fori_loop(0, lax.shift_right_logical(nr + 1, 1), pair,
                          jnp.int32(0))

            @pl.when(lax.bitwise_and(nr, 1) == 1)
            def _():
                s2wait(0)

            @pl.when(lax.bitwise_and(nr, 1) == 0)
            def _():
                s2wait(1)

    return gather_kernel(base_w, new_w, idx)


def kernel(input_ids, base_weight, new_weight):
    b, h = input_ids.shape
    d = base_weight.shape[1]
    idx = input_ids.reshape(-1).astype(jnp.int32)
    out = _gather_sc(base_weight, new_weight, idx, idx.shape[0], d)
    return out.reshape(b, h, d)


# speedup vs baseline: 2.1346x; 1.0103x over previous
import dataclasses
import functools

import jax
import jax.numpy as jnp
from jax import lax
from jax.experimental import pallas as pl
from jax.experimental.pallas import tpu as pltpu
from jax.experimental.pallas import tpu_sc as plsc

_NC = 2
_NS = 16
_NW = _NC * _NS
_W = 160
_K = 128
_VTXT = 100000


def _gather_sc(base_w, new_w, idx, n, d):
    b_per_w = n // _NW
    n_chunks = b_per_w // _W
    n_rounds = (b_per_w + _K - 1) // _K + 1
    assert n_chunks % 2 == 0 and n_chunks >= 4
    mesh = plsc.VectorSubcoreMesh(core_axis_name="c", subcore_axis_name="s")

    cparams = pltpu.CompilerParams()
    if "needs_layout_passes" in pltpu.CompilerParams.__dataclass_fields__:
        cparams = dataclasses.replace(cparams, needs_layout_passes=False)

    @functools.partial(
        pl.kernel,
        out_type=jax.ShapeDtypeStruct((n, d), jnp.float32),
        mesh=mesh,
        compiler_params=cparams,
        scratch_types=[
            pltpu.VMEM((b_per_w,), jnp.int32),
            pltpu.VMEM((_W,), jnp.int32),
            pltpu.VMEM((_W,), jnp.int32),
            pltpu.VMEM((_W, d), jnp.float32),
            pltpu.VMEM((_W, d), jnp.float32),
            pltpu.VMEM((n_rounds, _K), jnp.int32),
            pltpu.VMEM((n_rounds, _K), jnp.int32),
            pltpu.SMEM((2,), jnp.int32),
            pltpu.SemaphoreType.DMA,
            pltpu.SemaphoreType.DMA,
            pltpu.SemaphoreType.DMA,
            pltpu.SemaphoreType.DMA,
        ],
    )
    def gather_kernel(base_hbm, new_hbm, idx_hbm, out_hbm, idx_v,
                      midx0, midx1, rows0, rows1, fpos, fid, tot,
                      gsem0, gsem1, osem0, osem1):
        wid = lax.axis_index("s") * _NC + lax.axis_index("c")
        wbase = wid * b_per_w
        pltpu.sync_copy(idx_hbm.at[pl.ds(wbase, b_per_w)], idx_v)

        midx = (midx0, midx1)
        rows = (rows0, rows1)
        gsem = (gsem0, gsem1)
        osem = (osem0, osem1)

        iota = lax.iota(jnp.int32, 16)
        tot[0] = jnp.int32(0)

        def compute(c, p):
            cnt = tot[0]
            for v in range(_W // 16):
                ids = idx_v[pl.ds(c * _W + v * 16, 16)]
                m = ids >= _VTXT
                m32 = m.astype(jnp.int32)
                midx[p][pl.ds(v * 16, 16)] = ids - m32 * _VTXT
                pos = plsc.cumsum(m32) - 1 + cnt
                row = lax.shift_right_logical(pos, 7)
                col = lax.bitwise_and(pos, 127)
                gp = wbase + c * _W + v * 16 + iota
                plsc.store_scatter(fpos, [row, col], gp, mask=m)
                plsc.store_scatter(fid, [row, col], ids - _VTXT, mask=m)
                cnt = cnt + jnp.sum(m32)
            tot[0] = cnt

        def gstart(p):
            pltpu.async_copy(base_hbm.at[midx[p]], rows[p], gsem[p])

        def gwait(p):
            pltpu.make_async_copy(base_hbm.at[midx[p]], rows[p],
                                  gsem[p]).wait()

        def ostart(c, p):
            pltpu.async_copy(rows[p], out_hbm.at[pl.ds(wbase + c * _W, _W)],
                             osem[p])

        def owait(p):
            pltpu.make_async_copy(rows[p], out_hbm.at[pl.ds(wbase, _W)],
                                  osem[p]).wait()

        compute(0, 0)
        gstart(0)
        compute(1, 1)
        gstart(1)
        gwait(0)
        ostart(0, 0)

        @pl.loop(1, n_chunks - 1, step=2)
        def _(c):
            owait(0)
            compute(c + 1, 0)
            gstart(0)
            gwait(1)
            ostart(c, 1)
            owait(1)
            compute(c + 2, 1)
            gstart(1)
            gwait(0)
            ostart(c + 1, 0)

        gwait(1)
        ostart(n_chunks - 1, 1)
        owait(0)
        owait(1)

        cnt = tot[0]

        @pl.when(cnt > 0)
        def _():
            zeros = jnp.zeros((16,), jnp.int32)
            e0p = plsc.load_gather(fpos, [zeros, zeros])
            e0i = plsc.load_gather(fid, [zeros, zeros])
            for v in range(_K // 16):
                flat = cnt + v * 16 + iota
                row = lax.shift_right_logical(flat, 7)
                col = lax.bitwise_and(flat, 127)
                plsc.store_scatter(fpos, [row, col], e0p)
                plsc.store_scatter(fid, [row, col], e0i)

            nr = lax.shift_right_logical(cnt + (_K - 1), 7)

            def g2start(r, p):
                pltpu.async_copy(new_hbm.at[fid.at[r]],
                                 rows[p].at[pl.ds(0, _K)], gsem[p])

            def g2wait(p):
                pltpu.make_async_copy(new_hbm.at[pl.ds(0, _K)],
                                      rows[p].at[pl.ds(0, _K)],
                                      gsem[p]).wait()

            def s2start(r, p):
                pltpu.async_copy(rows[p].at[pl.ds(0, _K)],
                                 out_hbm.at[fpos.at[r]], osem[p])

            def s2wait(p):
                pltpu.make_async_copy(rows[p].at[pl.ds(0, _K)],
                                      out_hbm.at[pl.ds(wbase, _K)],
                                      osem[p]).wait()

            g2start(0, 0)

            def pair(i, carry):
                r0 = 2 * i
                r1 = r0 + 1
                g2wait(0)
                s2start(r0, 0)

                @pl.when(i > 0)
                def _():
                    s2wait(1)

                @pl.when(r1 < nr)
                def _():
                    g2start(r1, 1)
                    g2wait(1)
                    s2wait(0)

                    @pl.when(r1 + 1 < nr)
                    def _():
                        g2start(r1 + 1, 0)

                    s2start(r1, 1)

                return carry

            lax.fori_loop(0, lax.shift_right_logical(nr + 1, 1), pair,
                          jnp.int32(0))

            @pl.when(lax.bitwise_and(nr, 1) == 1)
            def _():
                s2wait(0)

            @pl.when(lax.bitwise_and(nr, 1) == 0)
            def _():
                s2wait(1)

    return gather_kernel(base_w, new_w, idx)


def kernel(input_ids, base_weight, new_weight):
    b, h = input_ids.shape
    d = base_weight.shape[1]
    idx = input_ids.reshape(-1).astype(jnp.int32)
    out = _gather_sc(base_weight, new_weight, idx, idx.shape[0], d)
    return out.reshape(b, h, d)
